# baseline (speedup 1.0000x reference)
import jax
import jax.numpy as jnp
from jax.experimental import pallas as pl
from jax.experimental.pallas import tpu as pltpu

_NB = 6
_LA = 4


def _body(x_ref, y_ref, X_ref, y_hbm, o_hbm, bufs, rsem, wsem):
    n, h, w = y_hbm.shape
    ch = bufs.shape[1]
    per_shot = h // ch
    nchunks = n * per_shot

    def rd(k, b):
        i, half = k // per_shot, k % per_shot
        return pltpu.make_async_copy(
            y_hbm.at[i, pl.ds(half * ch, ch), :], bufs.at[b], rsem.at[b]
        )

    def wr(k, b):
        i, half = k // per_shot, k % per_shot
        return pltpu.make_async_copy(
            bufs.at[b], o_hbm.at[i, pl.ds(half * ch, ch), :], wsem.at[b]
        )

    for k in range(min(_LA, nchunks)):
        rd(k, k % _NB).start()
    cols = jax.lax.broadcasted_iota(jnp.int32, (1, w), 1)
    for k in range(nchunks):
        b = k % _NB
        rd(k, b).wait()
        i, half = k // per_shot, k % per_shot
        r_loc = y_ref[i] - half * ch

        @pl.when((r_loc >= 0) & (r_loc < ch))
        def _inject():
            row = bufs[b, pl.ds(r_loc, 1), :]
            bufs[b, pl.ds(r_loc, 1), :] = row + jnp.where(
                cols == x_ref[i], X_ref[0], 0.0
            )

        wr(k, b).start()
        nk = k + _LA
        if nk < nchunks:
            if nk - _NB >= 0:
                wr(nk - _NB, nk % _NB).wait()
            rd(nk, nk % _NB).start()
    for k in range(max(nchunks - _NB, 0), nchunks):
        wr(k, k % _NB).wait()


def kernel(Y, X, x, y):
    n, h, w = Y.shape
    return pl.pallas_call(
        _body,
        in_specs=[
            pl.BlockSpec(memory_space=pltpu.SMEM),
            pl.BlockSpec(memory_space=pltpu.SMEM),
            pl.BlockSpec(memory_space=pltpu.SMEM),
            pl.BlockSpec(memory_space=pl.ANY),
        ],
        out_specs=pl.BlockSpec(memory_space=pl.ANY),
        out_shape=jax.ShapeDtypeStruct(Y.shape, Y.dtype),
        scratch_shapes=[
            pltpu.VMEM((_NB, h // 2, w), jnp.float32),
            pltpu.SemaphoreType.DMA((_NB,)),
            pltpu.SemaphoreType.DMA((_NB,)),
        ],
        compiler_params=pltpu.CompilerParams(
            vmem_limit_bytes=58 * 1024 * 1024,
        ),
    )(x, y, X, Y)


# speedup vs baseline: 1.0053x; 1.0053x over previous
import jax
import jax.numpy as jnp
from jax.experimental import pallas as pl
from jax.experimental.pallas import tpu as pltpu

_NB = 3
_LA = 3


def _body(x_ref, y_ref, X_ref, y_hbm, o_hbm, bufs, rsem, wsem):
    n, h, w = y_hbm.shape
    ch = bufs.shape[1]
    per_shot = h // ch
    nchunks = n * per_shot

    def rd(k, b):
        i, half = k // per_shot, k % per_shot
        return pltpu.make_async_copy(
            y_hbm.at[i, pl.ds(half * ch, ch), :], bufs.at[b], rsem.at[b]
        )

    def wr(k, b):
        i, half = k // per_shot, k % per_shot
        return pltpu.make_async_copy(
            bufs.at[b], o_hbm.at[i, pl.ds(half * ch, ch), :], wsem.at[b]
        )

    for k in range(min(_LA, nchunks)):
        rd(k, k % _NB).start()
    cols = jax.lax.broadcasted_iota(jnp.int32, (1, w), 1)
    for k in range(nchunks):
        b = k % _NB
        rd(k, b).wait()
        i, half = k // per_shot, k % per_shot
        r_loc = y_ref[i] - half * ch

        @pl.when((r_loc >= 0) & (r_loc < ch))
        def _inject():
            row = bufs[b, pl.ds(r_loc, 1), :]
            bufs[b, pl.ds(r_loc, 1), :] = row + jnp.where(
                cols == x_ref[i], X_ref[0], 0.0
            )

        wr(k, b).start()
        nk = k + _LA
        if nk < nchunks:
            if nk - _NB >= 0:
                wr(nk - _NB, nk % _NB).wait()
            rd(nk, nk % _NB).start()
    for k in range(max(nchunks - _NB, 0), nchunks):
        wr(k, k % _NB).wait()


def kernel(Y, X, x, y):
    n, h, w = Y.shape
    return pl.pallas_call(
        _body,
        in_specs=[
            pl.BlockSpec(memory_space=pltpu.SMEM),
            pl.BlockSpec(memory_space=pltpu.SMEM),
            pl.BlockSpec(memory_space=pltpu.SMEM),
            pl.BlockSpec(memory_space=pl.ANY),
        ],
        out_specs=pl.BlockSpec(memory_space=pl.ANY),
        out_shape=jax.ShapeDtypeStruct(Y.shape, Y.dtype),
        scratch_shapes=[
            pltpu.VMEM((_NB, h, w), jnp.float32),
            pltpu.SemaphoreType.DMA((_NB,)),
            pltpu.SemaphoreType.DMA((_NB,)),
        ],
        compiler_params=pltpu.CompilerParams(
            vmem_limit_bytes=58 * 1024 * 1024,
        ),
    )(x, y, X, Y)
